# baseline (speedup 1.0000x reference)
import jax
import jax.numpy as jnp
from jax.experimental import pallas as pl
from jax.experimental.pallas import tpu as pltpu

BF = jnp.bfloat16


def _kernel_a(adj_ref, x_ref, w1_ref, b1_ref, w2_ref, b2_ref,
              g2_ref, adjbf_ref, g1_s):
    @pl.when(pl.program_id(0) == 0)
    def _init():
        g1_s[...] = (
            jnp.dot(x_ref[...], w1_ref[...], preferred_element_type=jnp.float32)
            + b1_ref[...]
        ).astype(BF)

    ab = adj_ref[...].astype(BF)
    adjbf_ref[...] = ab
    t = jnp.dot(ab, g1_s[...], preferred_element_type=jnp.float32)
    t = jnp.maximum(t, 0.0).astype(BF)
    g2_ref[...] = (
        jnp.dot(t, w2_ref[...], preferred_element_type=jnp.float32) + b2_ref[...]
    ).astype(BF)


def _kernel_b(adj_ref, g2_ref, w3_ref, b3_ref, wp1_ref, bp1_ref, wp2_ref,
              bp2_ref, wp3_ref, bp3_ref, o_ref, g3_s, acc_s, *, bm):
    p = pl.program_id(0)
    i = pl.program_id(1)
    ng = pl.num_programs(1) - 1
    cur = jax.lax.rem(i, 2)
    prev = 1 - cur

    @pl.when(jnp.logical_and(p == 0, i < ng))
    def _dot_l2():
        acc_s[pl.ds(cur, 1)] = jnp.dot(
            adj_ref[...], g2_ref[...], preferred_element_type=jnp.float32
        )[None]

    @pl.when(jnp.logical_and(p == 1, i < ng))
    def _dot_l3():
        acc_s[pl.ds(cur, 1)] = jnp.dot(
            adj_ref[...], g3_s[...], preferred_element_type=jnp.float32
        )[None]

    @pl.when(jnp.logical_and(p == 0, i > 0))
    def _epi_l2():
        t = jnp.maximum(acc_s[pl.ds(prev, 1)][0], 0.0).astype(BF)
        g3_s[pl.ds((i - 1) * bm, bm), :] = (
            jnp.dot(t, w3_ref[...], preferred_element_type=jnp.float32)
            + b3_ref[...]
        ).astype(BF)

    @pl.when(jnp.logical_and(p == 1, i > 0))
    def _epi_head():
        t = jnp.maximum(acc_s[pl.ds(prev, 1)][0], 0.0).astype(BF)
        t = jnp.dot(t, wp1_ref[...], preferred_element_type=jnp.float32) + bp1_ref[...]
        t = jnp.maximum(t, 0.0).astype(BF)
        t = jnp.dot(t, wp2_ref[...], preferred_element_type=jnp.float32) + bp2_ref[...]
        t = jnp.maximum(t, 0.0).astype(BF)
        o_ref[0] = (
            jnp.dot(t, wp3_ref[...], preferred_element_type=jnp.float32)
            + bp3_ref[...]
        )


def kernel(x, adj, W1, b1, W2, b2, W3, b3, Wp1, bp1, Wp2, bp2, Wp3, bp3):
    import functools

    n, f = x.shape
    bm = 1000 if n % 1000 == 0 else n
    grid = n // bm
    bm1 = 400 if n % 400 == 0 else n
    grid1 = n // bm1

    xb = x.astype(BF)
    w1, w2, w3 = W1.astype(BF), W2.astype(BF), W3.astype(BF)
    wp1, wp2, wp3 = Wp1.astype(BF), Wp2.astype(BF), Wp3.astype(BF)
    b1r, b2r, b3r = b1.reshape(1, -1), b2.reshape(1, -1), b3.reshape(1, -1)
    bp1r, bp2r = bp1.reshape(1, -1), bp2.reshape(1, -1)
    bp3r = bp3.reshape(1, -1)
    f1, f2, f3 = w1.shape[1], w2.shape[1], w3.shape[1]

    g2, adj_bf = pl.pallas_call(
        _kernel_a,
        grid=(grid1,),
        in_specs=[
            pl.BlockSpec((bm1, n), lambda i: (i, 0)),
            pl.BlockSpec((n, f), lambda i: (0, 0)),
            pl.BlockSpec(w1.shape, lambda i: (0, 0)),
            pl.BlockSpec(b1r.shape, lambda i: (0, 0)),
            pl.BlockSpec(w2.shape, lambda i: (0, 0)),
            pl.BlockSpec(b2r.shape, lambda i: (0, 0)),
        ],
        out_specs=[
            pl.BlockSpec((bm1, f2), lambda i: (i, 0)),
            pl.BlockSpec((bm1, n), lambda i: (i, 0)),
        ],
        out_shape=[
            jax.ShapeDtypeStruct((n, f2), BF),
            jax.ShapeDtypeStruct((n, n), BF),
        ],
        scratch_shapes=[pltpu.VMEM((n, f1), BF)],
        compiler_params=pltpu.CompilerParams(vmem_limit_bytes=100 * 1024 * 1024),
    )(adj, xb, w1, b1r, w2, b2r)

    out = pl.pallas_call(
        functools.partial(_kernel_b, bm=bm),
        grid=(2, grid + 1),
        in_specs=[
            pl.BlockSpec((bm, n), lambda p, i: (jnp.minimum(i, grid - 1), 0)),
            pl.BlockSpec((n, f2), lambda p, i: (0, 0)),
            pl.BlockSpec(w3.shape, lambda p, i: (0, 0)),
            pl.BlockSpec(b3r.shape, lambda p, i: (0, 0)),
            pl.BlockSpec(wp1.shape, lambda p, i: (0, 0)),
            pl.BlockSpec(bp1r.shape, lambda p, i: (0, 0)),
            pl.BlockSpec(wp2.shape, lambda p, i: (0, 0)),
            pl.BlockSpec(bp2r.shape, lambda p, i: (0, 0)),
            pl.BlockSpec(wp3.shape, lambda p, i: (0, 0)),
            pl.BlockSpec(bp3r.shape, lambda p, i: (0, 0)),
        ],
        out_specs=pl.BlockSpec(
            (1, bm, wp3.shape[1]),
            lambda p, i: (p, jnp.where(i == 0, grid, i - 1), 0),
        ),
        out_shape=jax.ShapeDtypeStruct((2, n + bm, wp3.shape[1]), jnp.float32),
        scratch_shapes=[
            pltpu.VMEM((n, f3), BF),
            pltpu.VMEM((2, bm, f2), jnp.float32),
        ],
        compiler_params=pltpu.CompilerParams(vmem_limit_bytes=100 * 1024 * 1024),
    )(adj_bf, g2, w3, b3r, wp1, bp1r, wp2, bp2r, wp3, bp3r)
    return out[1, :n]


# speedup vs baseline: 1.0239x; 1.0239x over previous
import jax
import jax.numpy as jnp
from jax.experimental import pallas as pl
from jax.experimental.pallas import tpu as pltpu

BF = jnp.bfloat16


def _kernel_a(adj_ref, x_ref, w1_ref, b1_ref, w2_ref, b2_ref,
              g2_ref, adjl_ref, adjr_ref, g1_s, *, kh):
    @pl.when(pl.program_id(0) == 0)
    def _init():
        g1_s[...] = (
            jnp.dot(x_ref[...], w1_ref[...], preferred_element_type=jnp.float32)
            + b1_ref[...]
        ).astype(BF)

    ab = adj_ref[...].astype(BF)
    adjl_ref[...] = ab[:, :kh]
    adjr_ref[...] = ab[:, kh:]
    t = jnp.dot(ab, g1_s[...], preferred_element_type=jnp.float32)
    t = jnp.maximum(t, 0.0).astype(BF)
    g2_ref[...] = (
        jnp.dot(t, w2_ref[...], preferred_element_type=jnp.float32) + b2_ref[...]
    ).astype(BF)


def _kernel_b(adjl_ref, adjr_ref, g2_ref, w3_ref, b3_ref, wp1_ref, bp1_ref,
              wp2_ref, bp2_ref, wp3_ref, bp3_ref, o_ref, g3_s, *, bm, kh):
    p = pl.program_id(0)
    i = pl.program_id(1)

    @pl.when(p == 0)
    def _layer2():
        g2v = g2_ref[...]
        t = (jnp.dot(adjl_ref[...], g2v[:kh], preferred_element_type=jnp.float32)
             + jnp.dot(adjr_ref[...], g2v[kh:], preferred_element_type=jnp.float32))
        t = jnp.maximum(t, 0.0).astype(BF)
        g3_s[pl.ds(i * bm, bm), :] = (
            jnp.dot(t, w3_ref[...], preferred_element_type=jnp.float32)
            + b3_ref[...]
        ).astype(BF)

    @pl.when(p == 1)
    def _layer3_head():
        g3v = g3_s[...]
        t = (jnp.dot(adjl_ref[...], g3v[:kh], preferred_element_type=jnp.float32)
             + jnp.dot(adjr_ref[...], g3v[kh:], preferred_element_type=jnp.float32))
        t = jnp.maximum(t, 0.0).astype(BF)
        t = jnp.dot(t, wp1_ref[...], preferred_element_type=jnp.float32) + bp1_ref[...]
        t = jnp.maximum(t, 0.0).astype(BF)
        t = jnp.dot(t, wp2_ref[...], preferred_element_type=jnp.float32) + bp2_ref[...]
        t = jnp.maximum(t, 0.0).astype(BF)
        o_ref[...] = (
            jnp.dot(t, wp3_ref[...], preferred_element_type=jnp.float32)
            + bp3_ref[...]
        )


def kernel(x, adj, W1, b1, W2, b2, W3, b3, Wp1, bp1, Wp2, bp2, Wp3, bp3):
    import functools

    n, f = x.shape
    bm = 1000 if n % 1000 == 0 else n
    grid = n // bm
    bm1 = 400 if n % 400 == 0 else n
    grid1 = n // bm1

    xb = x.astype(BF)
    w1, w2, w3 = W1.astype(BF), W2.astype(BF), W3.astype(BF)
    wp1, wp2, wp3 = Wp1.astype(BF), Wp2.astype(BF), Wp3.astype(BF)
    b1r, b2r, b3r = b1.reshape(1, -1), b2.reshape(1, -1), b3.reshape(1, -1)
    bp1r, bp2r = bp1.reshape(1, -1), bp2.reshape(1, -1)
    bp3r = bp3.reshape(1, -1)
    f1, f2, f3 = w1.shape[1], w2.shape[1], w3.shape[1]

    kh = n // 2
    g2, adj_l, adj_r = pl.pallas_call(
        functools.partial(_kernel_a, kh=kh),
        grid=(grid1,),
        in_specs=[
            pl.BlockSpec((bm1, n), lambda i: (i, 0)),
            pl.BlockSpec((n, f), lambda i: (0, 0)),
            pl.BlockSpec(w1.shape, lambda i: (0, 0)),
            pl.BlockSpec(b1r.shape, lambda i: (0, 0)),
            pl.BlockSpec(w2.shape, lambda i: (0, 0)),
            pl.BlockSpec(b2r.shape, lambda i: (0, 0)),
        ],
        out_specs=[
            pl.BlockSpec((bm1, f2), lambda i: (i, 0)),
            pl.BlockSpec((bm1, kh), lambda i: (i, 0)),
            pl.BlockSpec((bm1, kh), lambda i: (i, 0)),
        ],
        out_shape=[
            jax.ShapeDtypeStruct((n, f2), BF),
            jax.ShapeDtypeStruct((n, kh), BF),
            jax.ShapeDtypeStruct((n, kh), BF),
        ],
        scratch_shapes=[pltpu.VMEM((n, f1), BF)],
        compiler_params=pltpu.CompilerParams(vmem_limit_bytes=100 * 1024 * 1024),
    )(adj, xb, w1, b1r, w2, b2r)

    out = pl.pallas_call(
        functools.partial(_kernel_b, bm=bm, kh=kh),
        grid=(2, grid),
        in_specs=[
            pl.BlockSpec((bm, kh), lambda p, i: (i, 0)),
            pl.BlockSpec((bm, kh), lambda p, i: (i, 0)),
            pl.BlockSpec((n, f2), lambda p, i: (0, 0)),
            pl.BlockSpec(w3.shape, lambda p, i: (0, 0)),
            pl.BlockSpec(b3r.shape, lambda p, i: (0, 0)),
            pl.BlockSpec(wp1.shape, lambda p, i: (0, 0)),
            pl.BlockSpec(bp1r.shape, lambda p, i: (0, 0)),
            pl.BlockSpec(wp2.shape, lambda p, i: (0, 0)),
            pl.BlockSpec(bp2r.shape, lambda p, i: (0, 0)),
            pl.BlockSpec(wp3.shape, lambda p, i: (0, 0)),
            pl.BlockSpec(bp3r.shape, lambda p, i: (0, 0)),
        ],
        out_specs=pl.BlockSpec((bm, wp3.shape[1]),
                               lambda p, i: (i * p + (1 - p) * grid, 0)),
        out_shape=jax.ShapeDtypeStruct((n + bm, wp3.shape[1]), jnp.float32),
        scratch_shapes=[pltpu.VMEM((n, f3), BF)],
    )(adj_l, adj_r, g2, w3, b3r, wp1, bp1r, wp2, bp2r, wp3, bp3r)
    return out[:n]
